# baseline (speedup 1.0000x reference)
import functools

import jax
import jax.numpy as jnp
from jax import lax
from jax.experimental import pallas as pl
from jax.experimental.pallas import tpu as pltpu
from jax.experimental.pallas import tpu_sc as plsc

N = 10000
E = 320000
F_IN = 128

NC = 2
NS = 16
NW = NC * NS

CHUNK = 128
NCHUNK = 80
NPAIR = NCHUNK // 2
EPW_PAD = NCHUNK * CHUNK
E_PAD = NW * EPW_PAD
ACC_N = N + 8
RPS = 624
RPS_TAIL = N - NS * RPS

DEG_PAD = 10240
DEG_W = 8
RPSD = DEG_PAD // NS

_mesh = plsc.VectorSubcoreMesh(
    core_axis_name="c", subcore_axis_name="s", num_cores=NC, num_subcores=NS)

_sc_params = pltpu.CompilerParams(use_tc_tiling_on_sc=False)


@functools.partial(
    pl.kernel,
    out_type=jax.ShapeDtypeStruct((NC, DEG_PAD, DEG_W), jnp.float32),
    mesh=_mesh,
    scratch_types=[
        pltpu.VMEM((NCHUNK, CHUNK), jnp.int32),
        pltpu.VMEM((CHUNK, DEG_W), jnp.float32),
        pltpu.VMEM_SHARED((DEG_PAD, DEG_W), jnp.float32),
    ],
    compiler_params=_sc_params,
)
def _deg_kernel(dst_hbm, ones_hbm, deg_out, idx_v, ones_v, acc_sh):
    cid = lax.axis_index("c")
    sid = lax.axis_index("s")
    wid = sid * NC + cid
    pltpu.sync_copy(dst_hbm.at[wid], idx_v)
    pltpu.sync_copy(ones_hbm.at[pl.ds(0, CHUNK)], ones_v)
    pltpu.sync_copy(ones_hbm.at[pl.ds(sid * RPSD, RPSD)],
                    acc_sh.at[pl.ds(sid * RPSD, RPSD)])
    plsc.subcore_barrier()

    def body(g, carry):
        pltpu.sync_copy(ones_v, acc_sh.at[idx_v.at[g]], add=True)
        return carry

    lax.fori_loop(0, NCHUNK, body, 0)
    plsc.subcore_barrier()
    pltpu.sync_copy(acc_sh.at[pl.ds(sid * RPSD, RPSD)],
                    deg_out.at[cid, pl.ds(sid * RPSD, RPSD)])


def _make_edge_kernel(d_feat):
    @functools.partial(
        pl.kernel,
        out_type=jax.ShapeDtypeStruct((NC, N, d_feat), jnp.float32),
        mesh=_mesh,
        scratch_types=[
            pltpu.VMEM((NCHUNK, CHUNK), jnp.int32),
            pltpu.VMEM((NCHUNK, CHUNK), jnp.int32),
            pltpu.VMEM((CHUNK, d_feat), jnp.float32),
            pltpu.VMEM((CHUNK, d_feat), jnp.float32),
            pltpu.VMEM_SHARED((ACC_N, d_feat), jnp.float32),
            pltpu.SemaphoreType.DMA,
            pltpu.SemaphoreType.DMA,
        ],
        compiler_params=_sc_params,
    )
    def edge_kernel(hs_hbm, src_hbm, dst_hbm, acc_out,
                    src_v, dst_v, rows0, rows1, acc_sh, sem0, sem1):
        cid = lax.axis_index("c")
        sid = lax.axis_index("s")
        wid = sid * NC + cid
        pltpu.sync_copy(src_hbm.at[wid], src_v)
        pltpu.sync_copy(dst_hbm.at[wid], dst_v)
        pltpu.sync_copy(hs_hbm.at[pl.ds(sid * RPS, RPS)],
                        acc_sh.at[pl.ds(sid * RPS, RPS)])

        @pl.when(sid == NS - 1)
        def _():
            pltpu.sync_copy(hs_hbm.at[pl.ds(NS * RPS, RPS_TAIL)],
                            acc_sh.at[pl.ds(NS * RPS, RPS_TAIL)])

        plsc.subcore_barrier()

        pltpu.async_copy(hs_hbm.at[src_v.at[0]], rows0, sem0)

        def body(p, carry):
            g0 = 2 * p
            g1 = g0 + 1
            pltpu.async_copy(hs_hbm.at[src_v.at[g1]], rows1, sem1)
            pltpu.make_async_copy(hs_hbm.at[src_v.at[g0]], rows0, sem0).wait()
            pltpu.sync_copy(rows0, acc_sh.at[dst_v.at[g0]], add=True)

            @pl.when(p < NPAIR - 1)
            def _():
                pltpu.async_copy(hs_hbm.at[src_v.at[g0 + 2]], rows0, sem0)

            pltpu.make_async_copy(hs_hbm.at[src_v.at[g1]], rows1, sem1).wait()
            pltpu.sync_copy(rows1, acc_sh.at[dst_v.at[g1]], add=True)
            return carry

        lax.fori_loop(0, NPAIR, body, 0)
        plsc.subcore_barrier()
        pltpu.sync_copy(acc_sh.at[pl.ds(sid * RPS, RPS)],
                        acc_out.at[cid, pl.ds(sid * RPS, RPS)])

        @pl.when(sid == NS - 1)
        def _():
            pltpu.sync_copy(acc_sh.at[pl.ds(NS * RPS, RPS_TAIL)],
                            acc_out.at[cid, pl.ds(NS * RPS, RPS_TAIL)])

    return edge_kernel


_edge64 = _make_edge_kernel(64)
_edge32 = _make_edge_kernel(32)


BN = 2000


def _dinv(deg_ref):
    return lax.rsqrt(deg_ref[0, :, :1] + deg_ref[1, :, :1] - 1.0)


def _tc_first(deg2, x, w0):
    def body(deg_ref, x_ref, w_ref, out_ref):
        dinv = _dinv(deg_ref)
        h = jnp.dot(x_ref[...], w_ref[...], preferred_element_type=jnp.float32)
        out_ref[...] = dinv * h

    d_out = w0.shape[1]
    return pl.pallas_call(
        body,
        grid=(N // BN,),
        in_specs=[
            pl.BlockSpec((NC, BN, DEG_W), lambda i: (0, i, 0)),
            pl.BlockSpec((BN, F_IN), lambda i: (i, 0)),
            pl.BlockSpec((F_IN, d_out), lambda i: (0, 0)),
        ],
        out_specs=pl.BlockSpec((BN, d_out), lambda i: (i, 0)),
        out_shape=jax.ShapeDtypeStruct((N, d_out), jnp.float32),
    )(deg2, x, w0)


def _tc_mid(deg2, acc, hs, w, b):
    d_in = hs.shape[1]
    d_out = w.shape[1]

    def body(deg_ref, acc_ref, hs_ref, w_ref, b_ref, out_ref):
        dinv = _dinv(deg_ref)
        a = acc_ref[0] + acc_ref[1] - hs_ref[...]
        h = jnp.maximum(dinv * a + b_ref[...], 0.0)
        hn = jnp.dot(h, w_ref[...], preferred_element_type=jnp.float32)
        out_ref[...] = dinv * hn

    return pl.pallas_call(
        body,
        grid=(N // BN,),
        in_specs=[
            pl.BlockSpec((NC, BN, DEG_W), lambda i: (0, i, 0)),
            pl.BlockSpec((NC, BN, d_in), lambda i: (0, i, 0)),
            pl.BlockSpec((BN, d_in), lambda i: (i, 0)),
            pl.BlockSpec((d_in, d_out), lambda i: (0, 0)),
            pl.BlockSpec((1, d_in), lambda i: (0, 0)),
        ],
        out_specs=pl.BlockSpec((BN, d_out), lambda i: (i, 0)),
        out_shape=jax.ShapeDtypeStruct((N, d_out), jnp.float32),
    )(deg2, acc, hs, w, b)


def _tc_last(deg2, acc, hs, b):
    d_in = hs.shape[1]

    def body(deg_ref, acc_ref, hs_ref, b_ref, out_ref):
        dinv = _dinv(deg_ref)
        a = acc_ref[0] + acc_ref[1] - hs_ref[...]
        out_ref[...] = jnp.maximum(dinv * a + b_ref[...], 0.0)

    return pl.pallas_call(
        body,
        grid=(N // BN,),
        in_specs=[
            pl.BlockSpec((NC, BN, DEG_W), lambda i: (0, i, 0)),
            pl.BlockSpec((NC, BN, d_in), lambda i: (0, i, 0)),
            pl.BlockSpec((BN, d_in), lambda i: (i, 0)),
            pl.BlockSpec((1, d_in), lambda i: (0, 0)),
        ],
        out_specs=pl.BlockSpec((BN, d_in), lambda i: (i, 0)),
        out_shape=jax.ShapeDtypeStruct((N, d_in), jnp.float32),
    )(deg2, acc, hs, b)


def kernel(x, edge_index, batch, W0, b0, W1, b1, W2, b2):
    pad_src = jnp.zeros((E_PAD - E,), jnp.int32)
    pad_dst = jnp.full((E_PAD - E,), N, jnp.int32)
    src = jnp.concatenate([edge_index[0], pad_src]).reshape(NW, NCHUNK, CHUNK)
    dst = jnp.concatenate([edge_index[1], pad_dst]).reshape(NW, NCHUNK, CHUNK)
    ones = jnp.ones((DEG_PAD, DEG_W), jnp.float32)

    deg2 = _deg_kernel(dst, ones)

    hs1 = _tc_first(deg2, x, W0)
    acc1 = _edge64(hs1, src, dst)
    hs2 = _tc_mid(deg2, acc1, hs1, W1, b0.reshape(1, -1))
    acc2 = _edge32(hs2, src, dst)
    hs3 = _tc_mid(deg2, acc2, hs2, W2, b1.reshape(1, -1))
    acc3 = _edge32(hs3, src, dst)
    return _tc_last(deg2, acc3, hs3, b2.reshape(1, -1))


# speedup vs baseline: 1.9490x; 1.9490x over previous
import functools

import jax
import jax.numpy as jnp
from jax import lax
from jax.experimental import pallas as pl
from jax.experimental.pallas import tpu as pltpu
from jax.experimental.pallas import tpu_sc as plsc

N = 10000
E = 320000
F_IN = 128

NC = 2
NS = 16
NW = NC * NS

CHUNK = 128
NCHUNK = 80
NPAIR = NCHUNK // 2
EPW = E // NW
EPW_PAD = NCHUNK * CHUNK
PADW = EPW_PAD - EPW
ACC_N = N + PADW
RPS = 624
RPS_TAIL = N - NS * RPS

DEG_PAD = 10240
DEG_W = 8
RPSD = DEG_PAD // NS

_mesh = plsc.VectorSubcoreMesh(
    core_axis_name="c", subcore_axis_name="s", num_cores=NC, num_subcores=NS)

_sc_params = pltpu.CompilerParams(use_tc_tiling_on_sc=False)


@functools.partial(
    pl.kernel,
    out_type=jax.ShapeDtypeStruct((NC, DEG_PAD, DEG_W), jnp.float32),
    mesh=_mesh,
    scratch_types=[
        pltpu.VMEM((NCHUNK, CHUNK), jnp.int32),
        pltpu.VMEM((CHUNK, DEG_W), jnp.float32),
        pltpu.VMEM_SHARED((DEG_PAD, DEG_W), jnp.float32),
    ],
    compiler_params=_sc_params,
)
def _deg_kernel(dst_hbm, ones_hbm, deg_out, idx_v, ones_v, acc_sh):
    cid = lax.axis_index("c")
    sid = lax.axis_index("s")
    wid = sid * NC + cid
    pltpu.sync_copy(dst_hbm.at[wid], idx_v)
    pltpu.sync_copy(ones_hbm.at[pl.ds(0, CHUNK)], ones_v)
    pltpu.sync_copy(ones_hbm.at[pl.ds(sid * RPSD, RPSD)],
                    acc_sh.at[pl.ds(sid * RPSD, RPSD)])
    plsc.subcore_barrier()

    def body(g, carry):
        pltpu.sync_copy(ones_v, acc_sh.at[idx_v.at[g]], add=True)
        return carry

    lax.fori_loop(0, NCHUNK, body, 0)
    plsc.subcore_barrier()
    pltpu.sync_copy(acc_sh.at[pl.ds(sid * RPSD, RPSD)],
                    deg_out.at[cid, pl.ds(sid * RPSD, RPSD)])


def _make_edge_kernel(d_feat):
    @functools.partial(
        pl.kernel,
        out_type=jax.ShapeDtypeStruct((NC, N, d_feat), jnp.float32),
        mesh=_mesh,
        scratch_types=[
            pltpu.VMEM((NCHUNK, CHUNK), jnp.int32),
            pltpu.VMEM((NCHUNK, CHUNK), jnp.int32),
            pltpu.VMEM((CHUNK, d_feat), jnp.float32),
            pltpu.VMEM((CHUNK, d_feat), jnp.float32),
            pltpu.VMEM_SHARED((ACC_N, d_feat), jnp.float32),
            pltpu.SemaphoreType.DMA,
            pltpu.SemaphoreType.DMA,
        ],
        compiler_params=_sc_params,
    )
    def edge_kernel(hs_hbm, src_hbm, dst_hbm, acc_out,
                    src_v, dst_v, rows0, rows1, acc_sh, sem0, sem1):
        cid = lax.axis_index("c")
        sid = lax.axis_index("s")
        wid = sid * NC + cid
        pltpu.sync_copy(src_hbm.at[wid], src_v)
        pltpu.sync_copy(dst_hbm.at[wid], dst_v)
        pltpu.sync_copy(hs_hbm.at[pl.ds(sid * RPS, RPS)],
                        acc_sh.at[pl.ds(sid * RPS, RPS)])

        @pl.when(sid == NS - 1)
        def _():
            pltpu.sync_copy(hs_hbm.at[pl.ds(NS * RPS, RPS_TAIL)],
                            acc_sh.at[pl.ds(NS * RPS, RPS_TAIL)])

        plsc.subcore_barrier()

        pltpu.async_copy(hs_hbm.at[src_v.at[0]], rows0, sem0)

        def body(p, carry):
            g0 = 2 * p
            g1 = g0 + 1
            pltpu.async_copy(hs_hbm.at[src_v.at[g1]], rows1, sem1)
            pltpu.make_async_copy(hs_hbm.at[src_v.at[g0]], rows0, sem0).wait()
            pltpu.sync_copy(rows0, acc_sh.at[dst_v.at[g0]], add=True)

            @pl.when(p < NPAIR - 1)
            def _():
                pltpu.async_copy(hs_hbm.at[src_v.at[g0 + 2]], rows0, sem0)

            pltpu.make_async_copy(hs_hbm.at[src_v.at[g1]], rows1, sem1).wait()
            pltpu.sync_copy(rows1, acc_sh.at[dst_v.at[g1]], add=True)
            return carry

        lax.fori_loop(0, NPAIR, body, 0)
        plsc.subcore_barrier()
        pltpu.sync_copy(acc_sh.at[pl.ds(sid * RPS, RPS)],
                        acc_out.at[cid, pl.ds(sid * RPS, RPS)])

        @pl.when(sid == NS - 1)
        def _():
            pltpu.sync_copy(acc_sh.at[pl.ds(NS * RPS, RPS_TAIL)],
                            acc_out.at[cid, pl.ds(NS * RPS, RPS_TAIL)])

    return edge_kernel


_edge64 = _make_edge_kernel(64)
_edge32 = _make_edge_kernel(32)


BN = 2000


def _dinv(deg_ref):
    return lax.rsqrt(deg_ref[0, :, :1] + deg_ref[1, :, :1] - 1.0)


def _tc_first(deg2, x, w0):
    def body(deg_ref, x_ref, w_ref, out_ref):
        dinv = _dinv(deg_ref)
        h = jnp.dot(x_ref[...], w_ref[...], preferred_element_type=jnp.float32)
        out_ref[...] = dinv * h

    d_out = w0.shape[1]
    return pl.pallas_call(
        body,
        grid=(N // BN,),
        in_specs=[
            pl.BlockSpec((NC, BN, DEG_W), lambda i: (0, i, 0)),
            pl.BlockSpec((BN, F_IN), lambda i: (i, 0)),
            pl.BlockSpec((F_IN, d_out), lambda i: (0, 0)),
        ],
        out_specs=pl.BlockSpec((BN, d_out), lambda i: (i, 0)),
        out_shape=jax.ShapeDtypeStruct((N, d_out), jnp.float32),
    )(deg2, x, w0)


def _tc_mid(deg2, acc, hs, w, b):
    d_in = hs.shape[1]
    d_out = w.shape[1]

    def body(deg_ref, acc_ref, hs_ref, w_ref, b_ref, out_ref):
        dinv = _dinv(deg_ref)
        a = acc_ref[0] + acc_ref[1] - hs_ref[...]
        h = jnp.maximum(dinv * a + b_ref[...], 0.0)
        hn = jnp.dot(h, w_ref[...], preferred_element_type=jnp.float32)
        out_ref[...] = dinv * hn

    return pl.pallas_call(
        body,
        grid=(N // BN,),
        in_specs=[
            pl.BlockSpec((NC, BN, DEG_W), lambda i: (0, i, 0)),
            pl.BlockSpec((NC, BN, d_in), lambda i: (0, i, 0)),
            pl.BlockSpec((BN, d_in), lambda i: (i, 0)),
            pl.BlockSpec((d_in, d_out), lambda i: (0, 0)),
            pl.BlockSpec((1, d_in), lambda i: (0, 0)),
        ],
        out_specs=pl.BlockSpec((BN, d_out), lambda i: (i, 0)),
        out_shape=jax.ShapeDtypeStruct((N, d_out), jnp.float32),
    )(deg2, acc, hs, w, b)


def _tc_last(deg2, acc, hs, b):
    d_in = hs.shape[1]

    def body(deg_ref, acc_ref, hs_ref, b_ref, out_ref):
        dinv = _dinv(deg_ref)
        a = acc_ref[0] + acc_ref[1] - hs_ref[...]
        out_ref[...] = jnp.maximum(dinv * a + b_ref[...], 0.0)

    return pl.pallas_call(
        body,
        grid=(N // BN,),
        in_specs=[
            pl.BlockSpec((NC, BN, DEG_W), lambda i: (0, i, 0)),
            pl.BlockSpec((NC, BN, d_in), lambda i: (0, i, 0)),
            pl.BlockSpec((BN, d_in), lambda i: (i, 0)),
            pl.BlockSpec((1, d_in), lambda i: (0, 0)),
        ],
        out_specs=pl.BlockSpec((BN, d_in), lambda i: (i, 0)),
        out_shape=jax.ShapeDtypeStruct((N, d_in), jnp.float32),
    )(deg2, acc, hs, b)


def kernel(x, edge_index, batch, W0, b0, W1, b1, W2, b2):
    pad_src = jnp.broadcast_to(jnp.arange(PADW, dtype=jnp.int32), (NW, PADW))
    pad_dst = jnp.broadcast_to(N + jnp.arange(PADW, dtype=jnp.int32), (NW, PADW))
    src = jnp.concatenate([edge_index[0].reshape(NW, EPW), pad_src],
                          axis=1).reshape(NW, NCHUNK, CHUNK)
    dst = jnp.concatenate([edge_index[1].reshape(NW, EPW), pad_dst],
                          axis=1).reshape(NW, NCHUNK, CHUNK)
    ones = jnp.ones((DEG_PAD, DEG_W), jnp.float32)

    deg2 = _deg_kernel(dst, ones)

    hs1 = _tc_first(deg2, x, W0)
    acc1 = _edge64(hs1, src, dst)
    hs2 = _tc_mid(deg2, acc1, hs1, W1, b0.reshape(1, -1))
    acc2 = _edge32(hs2, src, dst)
    hs3 = _tc_mid(deg2, acc2, hs2, W2, b1.reshape(1, -1))
    acc3 = _edge32(hs3, src, dst)
    return _tc_last(deg2, acc3, hs3, b2.reshape(1, -1))
